# Initial kernel scaffold; baseline (speedup 1.0000x reference)
import jax
import jax.numpy as jnp
from jax.experimental import pallas as pl


def kernel(t, table):
    raise NotImplementedError("write your pallas kernel here")


# speedup vs baseline: 4.7827x; 4.7827x over previous
import functools

import jax
import jax.numpy as jnp
from jax import lax
from jax.experimental import pallas as pl
from jax.experimental.pallas import tpu as pltpu
from jax.experimental.pallas import tpu_sc as plsc

CONTEXT = 8192
EMBED = 64
B_TOTAL = 4096 * 200
NC, NS = 2, 16
NW = NC * NS
B_PER_W = B_TOTAL // NW
CHUNK = 800
N_CHUNKS = B_PER_W // CHUNK


def _gather_body(t_hbm, table_hbm, out_hbm, idx_v, rows_v, sem):
    wid = lax.axis_index("s") * NC + lax.axis_index("c")
    base = wid * B_PER_W

    def chunk(i, carry):
        off = base + i * CHUNK
        pltpu.sync_copy(t_hbm.at[pl.ds(off, CHUNK)], idx_v)
        pltpu.async_copy(table_hbm.at[idx_v], rows_v, sem).wait()
        pltpu.sync_copy(rows_v, out_hbm.at[pl.ds(off, CHUNK)])
        return carry

    lax.fori_loop(0, N_CHUNKS, chunk, 0)


@jax.jit
def _lookup(t_flat, table):
    mesh = plsc.VectorSubcoreMesh(core_axis_name="c", subcore_axis_name="s")
    f = pl.kernel(
        _gather_body,
        out_type=jax.ShapeDtypeStruct((B_TOTAL, EMBED), jnp.float32),
        mesh=mesh,
        scratch_types=[
            pltpu.VMEM((CHUNK,), jnp.int32),
            pltpu.VMEM((CHUNK, EMBED), jnp.float32),
            pltpu.SemaphoreType.DMA,
        ],
        compiler_params=pltpu.CompilerParams(use_tc_tiling_on_sc=False),
    )
    return f(t_flat, table)


def kernel(t, table):
    out = _lookup(t.reshape(-1).astype(jnp.int32), table)
    return out.reshape(t.shape + (EMBED,))


# speedup vs baseline: 4.9689x; 1.0389x over previous
import jax
import jax.numpy as jnp
from jax import lax
from jax.experimental import pallas as pl
from jax.experimental.pallas import tpu as pltpu
from jax.experimental.pallas import tpu_sc as plsc

CONTEXT = 8192
EMBED = 64
B_TOTAL = 4096 * 200
NC, NS = 2, 16
NW = NC * NS
B_PER_W = B_TOTAL // NW
CHUNK = 800
N_CHUNKS = B_PER_W // CHUNK


def _gather_body(t_hbm, table_hbm, out_hbm, idx_v, rows_v,
                 gsem0, gsem1, ssem0, ssem1):
    wid = lax.axis_index("s") * NC + lax.axis_index("c")
    base = wid * B_PER_W
    gsems = (gsem0, gsem1)
    ssems = (ssem0, ssem1)

    pltpu.sync_copy(
        t_hbm.at[pl.ds(base, B_PER_W)],
        idx_v.at[pl.ds(0, B_PER_W)],
    )

    def gather(c, b):
        return pltpu.make_async_copy(
            table_hbm.at[idx_v.at[pl.ds(c * CHUNK, CHUNK)]],
            rows_v.at[b],
            gsems[b],
        )

    def store(c, b):
        return pltpu.make_async_copy(
            rows_v.at[b],
            out_hbm.at[pl.ds(base + c * CHUNK, CHUNK)],
            ssems[b],
        )

    def step_half(i, b):
        nb = 1 - b

        @pl.when(i < N_CHUNKS - 1)
        def _():
            @pl.when(i >= 1)
            def _():
                store(0, nb).wait()
            gather(i + 1, nb).start()

        gather(0, b).wait()
        store(i, b).start()

    gather(0, 0).start()

    def pair(j, carry):
        step_half(2 * j, 0)
        step_half(2 * j + 1, 1)
        return carry

    lax.fori_loop(0, N_CHUNKS // 2, pair, 0)
    store(0, 0).wait()
    store(0, 1).wait()


@jax.jit
def _lookup(t_flat, table):
    mesh = plsc.VectorSubcoreMesh(core_axis_name="c", subcore_axis_name="s")
    f = pl.kernel(
        _gather_body,
        out_type=jax.ShapeDtypeStruct((B_TOTAL, EMBED), jnp.float32),
        mesh=mesh,
        scratch_types=[
            pltpu.VMEM((B_PER_W,), jnp.int32),
            pltpu.VMEM((2, CHUNK, EMBED), jnp.float32),
            pltpu.SemaphoreType.DMA,
            pltpu.SemaphoreType.DMA,
            pltpu.SemaphoreType.DMA,
            pltpu.SemaphoreType.DMA,
        ],
        compiler_params=pltpu.CompilerParams(use_tc_tiling_on_sc=False),
    )
    return f(t_flat, table)


def kernel(t, table):
    out = _lookup(t.reshape(-1).astype(jnp.int32), table)
    return out.reshape(t.shape + (EMBED,))


# speedup vs baseline: 5.6191x; 1.1309x over previous
import jax
import jax.numpy as jnp
from jax import lax
from jax.experimental import pallas as pl
from jax.experimental.pallas import tpu as pltpu
from jax.experimental.pallas import tpu_sc as plsc

CONTEXT = 8192
EMBED = 64
B_TOTAL = 4096 * 200
NC, NS = 2, 16
NW = NC * NS
B_PER_W = B_TOTAL // NW
CHUNK = 512
N_CHUNKS = B_PER_W // CHUNK


def _gather_body(t_hbm, table_hbm, out_hbm, idx_v, rows_v, table_sh,
                 gsem0, gsem1, ssem0, ssem1):
    wid = lax.axis_index("s") * NC + lax.axis_index("c")
    sid = lax.axis_index("s")
    base = wid * B_PER_W
    gsems = (gsem0, gsem1)
    ssems = (ssem0, ssem1)

    @pl.when(sid == 0)
    def _():
        pltpu.sync_copy(table_hbm, table_sh)
    plsc.subcore_barrier()

    pltpu.sync_copy(
        t_hbm.at[pl.ds(base, B_PER_W)],
        idx_v.at[pl.ds(0, B_PER_W)],
    )

    def gather(c, b):
        return pltpu.make_async_copy(
            table_sh.at[idx_v.at[pl.ds(c * CHUNK, CHUNK)]],
            rows_v.at[b],
            gsems[b],
        )

    def store(c, b):
        return pltpu.make_async_copy(
            rows_v.at[b],
            out_hbm.at[pl.ds(base + c * CHUNK, CHUNK)],
            ssems[b],
        )

    def step_half(i, b):
        nb = 1 - b

        @pl.when(i < N_CHUNKS - 1)
        def _():
            @pl.when(i >= 1)
            def _():
                store(0, nb).wait()
            gather(i + 1, nb).start()

        gather(0, b).wait()
        store(i, b).start()

    gather(0, 0).start()

    def pair(j, carry):
        step_half(2 * j, 0)
        step_half(2 * j + 1, 1)
        return carry

    lax.fori_loop(0, N_CHUNKS // 2, pair, 0)
    store(0, 0).wait()
    store(0, 1).wait()


@jax.jit
def _lookup(t_flat, table):
    mesh = plsc.VectorSubcoreMesh(core_axis_name="c", subcore_axis_name="s")
    f = pl.kernel(
        _gather_body,
        out_type=jax.ShapeDtypeStruct((B_TOTAL, EMBED), jnp.float32),
        mesh=mesh,
        scratch_types=[
            pltpu.VMEM((B_PER_W,), jnp.int32),
            pltpu.VMEM((2, CHUNK, EMBED), jnp.float32),
            pltpu.VMEM_SHARED((CONTEXT, EMBED), jnp.float32),
            pltpu.SemaphoreType.DMA,
            pltpu.SemaphoreType.DMA,
            pltpu.SemaphoreType.DMA,
            pltpu.SemaphoreType.DMA,
        ],
        compiler_params=pltpu.CompilerParams(use_tc_tiling_on_sc=False),
    )
    return f(t_flat, table)


def kernel(t, table):
    out = _lookup(t.reshape(-1).astype(jnp.int32), table)
    return out.reshape(t.shape + (EMBED,))


# speedup vs baseline: 23.6761x; 4.2135x over previous
import jax
import jax.numpy as jnp
from jax import lax
from jax.experimental import pallas as pl
from jax.experimental.pallas import tpu as pltpu
from jax.experimental.pallas import tpu_sc as plsc

CONTEXT = 8192
EMBED = 64
NROW = 4096
NPOS = 200
NC, NS = 2, 16
NW = NC * NS
NDG = 8
DPG = EMBED // NDG
NPG = NW // NDG
PBLK = 8
NPB = NPOS // PBLK
RBLK = 256
NRB = NROW // RBLK
RGPC = RBLK // 128


def _gather_body(tP, tflat, out, table_v, idx0, idx1, out0, out1,
                 isem0, isem1, osem0, osem1):
    wid = lax.axis_index("s") * NC + lax.axis_index("c")
    dg = lax.rem(wid, NDG)
    pg = wid // NDG
    idx_bufs = (idx0, idx1)
    out_bufs = (out0, out1)
    isems = (isem0, isem1)
    osems = (osem0, osem1)

    for k in range(DPG):
        pltpu.sync_copy(
            tflat.at[pl.ds((dg * DPG + k) * CONTEXT, CONTEXT)],
            table_v.at[pl.ds(k * CONTEXT, CONTEXT)],
        )

    nchunks = NPB * NRB // NPG

    def coords(c):
        u = pg + NPG * c
        pb = u // NRB
        rg = lax.rem(u, NRB)
        return pb * PBLK, rg

    def idx_dma(c, b):
        p0, rg = coords(c)
        return pltpu.make_async_copy(
            tP.at[pl.ds(p0, PBLK), pl.ds(rg * RBLK, RBLK)],
            idx_bufs[b], isems[b])

    def out_dma(c, b, rgl):
        p0, rg = coords(c)
        return pltpu.make_async_copy(
            out_bufs[b].at[rgl],
            out.at[pl.ds(p0, PBLK), dg, rg * RGPC + rgl],
            osems[b])

    def compute(b):
        idx_v = idx_bufs[b]
        out_v = out_bufs[b]

        @plsc.parallel_loop(0, PBLK * (RBLK // 16), step=1, unroll=8)
        def nbody(n):
            pp = n // (RBLK // 16)
            j = lax.rem(n, RBLK // 16)
            rgl = j // 8
            base = lax.rem(j, 8) * 16
            iv = idx_v[pp, pl.ds(j * 16, 16)]
            for dd in range(DPG):
                out_v[rgl, pp, dd, pl.ds(base, 16)] = plsc.load_gather(
                    table_v.at[pl.ds(dd * CONTEXT, CONTEXT)], [iv])

    def half(c, b):
        idx_dma(c, b).wait()

        @pl.when(c >= 2)
        def _():
            out_dma(0, b, 0).wait()
            out_dma(0, b, 0).wait()

        compute(b)
        out_dma(c, b, 0).start()
        out_dma(c, b, 1).start()

        @pl.when(c + 2 < nchunks)
        def _():
            idx_dma(c + 2, b).start()

    idx_dma(0, 0).start()
    idx_dma(1, 1).start()

    def pair(j, carry):
        half(2 * j, 0)
        half(2 * j + 1, 1)
        return carry

    lax.fori_loop(0, nchunks // 2, pair, 0)
    out_dma(0, 0, 0).wait()
    out_dma(0, 0, 0).wait()
    out_dma(0, 1, 0).wait()
    out_dma(0, 1, 0).wait()


@jax.jit
def _lookup(tP, tflat):
    mesh = plsc.VectorSubcoreMesh(core_axis_name="c", subcore_axis_name="s")
    f = pl.kernel(
        _gather_body,
        out_type=jax.ShapeDtypeStruct(
            (NPOS, NDG, NROW // 128, DPG, 128), jnp.float32),
        mesh=mesh,
        scratch_types=[
            pltpu.VMEM((DPG * CONTEXT,), jnp.float32),
            pltpu.VMEM((PBLK, RBLK), jnp.int32),
            pltpu.VMEM((PBLK, RBLK), jnp.int32),
            pltpu.VMEM((RGPC, PBLK, DPG, 128), jnp.float32),
            pltpu.VMEM((RGPC, PBLK, DPG, 128), jnp.float32),
            pltpu.SemaphoreType.DMA,
            pltpu.SemaphoreType.DMA,
            pltpu.SemaphoreType.DMA,
            pltpu.SemaphoreType.DMA,
        ],
        compiler_params=pltpu.CompilerParams(
            use_tc_tiling_on_sc=False, needs_layout_passes=False),
    )
    return f(tP, tflat)


def kernel(t, table):
    out5 = _lookup(t.T, table.T.reshape(-1).astype(jnp.float32))
    return jnp.transpose(out5, (2, 4, 0, 1, 3)).reshape(NROW, NPOS, EMBED)


# speedup vs baseline: 23.8632x; 1.0079x over previous
import jax
import jax.numpy as jnp
from jax import lax
from jax.experimental import pallas as pl
from jax.experimental.pallas import tpu as pltpu
from jax.experimental.pallas import tpu_sc as plsc

CONTEXT = 8192
EMBED = 64
NROW = 4096
NPOS = 200
NC, NS = 2, 16
NW = NC * NS
NDG = 8
DPG = EMBED // NDG
NPG = NW // NDG
PBLK = 8
NPB = NPOS // PBLK
RBLK = 256
NRB = NROW // RBLK
RGPC = RBLK // 128


def _gather_body(tP, tflat, out, table_v, idx0, idx1, out0, out1,
                 isem0, isem1, osem0, osem1):
    wid = lax.axis_index("s") * NC + lax.axis_index("c")
    dg = lax.rem(wid, NDG)
    pg = wid // NDG
    idx_bufs = (idx0, idx1)
    out_bufs = (out0, out1)
    isems = (isem0, isem1)
    osems = (osem0, osem1)

    for k in range(DPG):
        pltpu.sync_copy(
            tflat.at[pl.ds((dg * DPG + k) * CONTEXT, CONTEXT)],
            table_v.at[pl.ds(k * CONTEXT, CONTEXT)],
        )

    nchunks = NPB * NRB // NPG

    def coords(c):
        u = pg + NPG * c
        pb = u // NRB
        rg = lax.rem(u, NRB)
        return pb * PBLK, rg

    def idx_dma(c, b):
        p0, rg = coords(c)
        return pltpu.make_async_copy(
            tP.at[pl.ds(p0, PBLK), pl.ds(rg * RBLK, RBLK)],
            idx_bufs[b], isems[b])

    def out_dma(c, b, rgl):
        p0, rg = coords(c)
        return pltpu.make_async_copy(
            out_bufs[b].at[rgl],
            out.at[pl.ds(p0, PBLK), dg, rg * RGPC + rgl],
            osems[b])

    def compute(b):
        idx_v = idx_bufs[b]
        out_v = out_bufs[b]

        @plsc.parallel_loop(0, PBLK * (RBLK // 16), step=1, unroll=16)
        def nbody(n):
            pp = n // (RBLK // 16)
            j = lax.rem(n, RBLK // 16)
            rgl = j // 8
            base = lax.rem(j, 8) * 16
            iv = idx_v[pp, pl.ds(j * 16, 16)]
            for dd in range(DPG):
                out_v[rgl, pp, dd, pl.ds(base, 16)] = plsc.load_gather(
                    table_v.at[pl.ds(dd * CONTEXT, CONTEXT)], [iv])

    def half(c, b):
        idx_dma(c, b).wait()

        @pl.when(c >= 2)
        def _():
            out_dma(0, b, 0).wait()
            out_dma(0, b, 0).wait()

        compute(b)
        out_dma(c, b, 0).start()
        out_dma(c, b, 1).start()

        @pl.when(c + 2 < nchunks)
        def _():
            idx_dma(c + 2, b).start()

    idx_dma(0, 0).start()
    idx_dma(1, 1).start()

    def pair(j, carry):
        half(2 * j, 0)
        half(2 * j + 1, 1)
        return carry

    lax.fori_loop(0, nchunks // 2, pair, 0)
    out_dma(0, 0, 0).wait()
    out_dma(0, 0, 0).wait()
    out_dma(0, 1, 0).wait()
    out_dma(0, 1, 0).wait()


@jax.jit
def _lookup(tP, tflat):
    mesh = plsc.VectorSubcoreMesh(core_axis_name="c", subcore_axis_name="s")
    f = pl.kernel(
        _gather_body,
        out_type=jax.ShapeDtypeStruct(
            (NPOS, NDG, NROW // 128, DPG, 128), jnp.float32),
        mesh=mesh,
        scratch_types=[
            pltpu.VMEM((DPG * CONTEXT,), jnp.float32),
            pltpu.VMEM((PBLK, RBLK), jnp.int32),
            pltpu.VMEM((PBLK, RBLK), jnp.int32),
            pltpu.VMEM((RGPC, PBLK, DPG, 128), jnp.float32),
            pltpu.VMEM((RGPC, PBLK, DPG, 128), jnp.float32),
            pltpu.SemaphoreType.DMA,
            pltpu.SemaphoreType.DMA,
            pltpu.SemaphoreType.DMA,
            pltpu.SemaphoreType.DMA,
        ],
        compiler_params=pltpu.CompilerParams(
            use_tc_tiling_on_sc=False, needs_layout_passes=False),
    )
    return f(tP, tflat)


def kernel(t, table):
    out5 = _lookup(t.T, table.T.reshape(-1).astype(jnp.float32))
    return jnp.transpose(out5, (2, 4, 0, 1, 3)).reshape(NROW, NPOS, EMBED)
